# Initial kernel scaffold; baseline (speedup 1.0000x reference)
import jax
import jax.numpy as jnp
from jax.experimental import pallas as pl


def kernel(node_values):
    raise NotImplementedError("write your pallas kernel here")


# speedup vs baseline: 115.0577x; 115.0577x over previous
import functools

import jax
import jax.numpy as jnp
from jax import lax
from jax.experimental import pallas as pl
from jax.experimental.pallas import tpu as pltpu
from jax.experimental.pallas import tpu_sc as plsc

N = 4_000_000
NUM_LEVELS = 4096
L = 16
NC = 2
NS = 16
NW = NC * NS
E = N // NW
CH = 5_000
NCHUNK = E // CH
FULL_VECS = CH // L
TAIL = CH - FULL_VECS * L
BUF = (CH + L - 1) // L * L
HBINS = NUM_LEVELS // NS
HVECS = NUM_LEVELS // L

_mesh = plsc.VectorSubcoreMesh(core_axis_name="c", subcore_axis_name="s")
_params = pltpu.CompilerParams(needs_layout_passes=False)


@functools.partial(
    pl.kernel,
    mesh=_mesh,
    out_type=jax.ShapeDtypeStruct((NC, NUM_LEVELS), jnp.float32),
    scratch_types=[
        pltpu.VMEM((BUF,), jnp.float32),
        pltpu.VMEM((NUM_LEVELS,), jnp.float32),
        pltpu.VMEM((HBINS,), jnp.float32),
        pltpu.VMEM((HBINS,), jnp.float32),
        pltpu.VMEM_SHARED((NS, NUM_LEVELS), jnp.float32),
    ],
    compiler_params=_params,
)
def _hist_kernel(vals_hbm, part_hbm, buf, hist, tmp, acc, shared):
    c = lax.axis_index("c")
    s = lax.axis_index("s")
    wid = s * NC + c
    base = wid * E

    zeros16 = jnp.zeros((L,), jnp.float32)
    ones16 = jnp.ones((L,), jnp.float32)
    tailmask = lax.iota(jnp.int32, L) < TAIL

    def zbody(i, carry):
        hist[pl.ds(i * L, L)] = zeros16
        return carry

    lax.fori_loop(0, HVECS, zbody, 0)

    def chunk_body(ch, carry):
        pltpu.sync_copy(vals_hbm.at[pl.ds(base + ch * CH, CH)],
                        buf.at[pl.ds(0, CH)])

        def vec_body(i, carry2):
            idx = buf[pl.ds(i * L, L)].astype(jnp.int32)
            plsc.addupdate_scatter(hist, [idx], ones16)
            return carry2

        lax.fori_loop(0, FULL_VECS, vec_body, 0)
        v = buf[pl.ds(FULL_VECS * L, L)]
        idx = jnp.clip(v.astype(jnp.int32), 0, NUM_LEVELS - 1)
        plsc.addupdate_scatter(hist, [idx], ones16, mask=tailmask)
        return carry

    lax.fori_loop(0, NCHUNK, chunk_body, 0)

    pltpu.sync_copy(hist, shared.at[s])
    plsc.subcore_barrier()

    def zacc(i, carry):
        acc[pl.ds(i * L, L)] = zeros16
        return carry

    lax.fori_loop(0, HBINS // L, zacc, 0)

    def row_body(j, carry):
        pltpu.sync_copy(shared.at[j, pl.ds(s * HBINS, HBINS)], tmp)

        def add_body(k, carry2):
            sl = pl.ds(k * L, L)
            acc[sl] = acc[sl] + tmp[sl]
            return carry2

        lax.fori_loop(0, HBINS // L, add_body, 0)
        return carry

    lax.fori_loop(0, NS, row_body, 0)
    pltpu.sync_copy(acc, part_hbm.at[c, pl.ds(s * HBINS, HBINS)])


@functools.partial(
    pl.kernel,
    mesh=_mesh,
    out_type=jax.ShapeDtypeStruct((N,), jnp.float32),
    scratch_types=[
        pltpu.VMEM((BUF,), jnp.float32),
        pltpu.VMEM((BUF,), jnp.float32),
        pltpu.VMEM((NUM_LEVELS,), jnp.float32),
        pltpu.VMEM((NUM_LEVELS,), jnp.float32),
    ],
    compiler_params=_params,
)
def _lookup_kernel(part_hbm, vals_hbm, out_hbm, buf, obuf, hist, h2):
    c = lax.axis_index("c")
    s = lax.axis_index("s")
    wid = s * NC + c
    base = wid * E

    pltpu.sync_copy(part_hbm.at[0], hist)
    pltpu.sync_copy(part_hbm.at[1], h2)

    def addmax_body(i, m):
        sl = pl.ds(i * L, L)
        hv = hist[sl] + h2[sl]
        hist[sl] = hv
        return jnp.maximum(m, hv)

    m = lax.fori_loop(0, HVECS, addmax_body, jnp.zeros((L,), jnp.float32))
    mx = lax.broadcast(jnp.max(m), (L,))
    recip = jnp.ones((L,), jnp.float32) / mx

    buf[pl.ds(BUF - L, L)] = jnp.zeros((L,), jnp.float32)

    def chunk_body(ch, carry):
        pltpu.sync_copy(vals_hbm.at[pl.ds(base + ch * CH, CH)],
                        buf.at[pl.ds(0, CH)])

        def vec_body(i, carry2):
            sl = pl.ds(i * L, L)
            idx = buf[sl].astype(jnp.int32)
            obuf[sl] = plsc.load_gather(hist, [idx]) * recip
            return carry2

        lax.fori_loop(0, FULL_VECS + 1, vec_body, 0)
        pltpu.sync_copy(obuf.at[pl.ds(0, CH)],
                        out_hbm.at[pl.ds(base + ch * CH, CH)])
        return carry

    lax.fori_loop(0, NCHUNK, chunk_body, 0)


def kernel(node_values):
    part = _hist_kernel(node_values)
    return _lookup_kernel(part, node_values)


# speedup vs baseline: 344.6356x; 2.9953x over previous
import functools

import jax
import jax.numpy as jnp
from jax import lax
from jax.experimental import pallas as pl
from jax.experimental.pallas import tpu as pltpu
from jax.experimental.pallas import tpu_sc as plsc

N = 4_000_000
NUM_LEVELS = 4096
L = 16
NC = 2
NS = 16
NW = NC * NS
E = N // NW
CH = 25_000
NCHUNK = E // CH
FULL_VECS = CH // L
TAIL = CH - FULL_VECS * L
BUF = (CH + L - 1) // L * L
HBINS = NUM_LEVELS // NS
HVECS = NUM_LEVELS // L
UNROLL = 8

_mesh = plsc.VectorSubcoreMesh(core_axis_name="c", subcore_axis_name="s")
_params = pltpu.CompilerParams(needs_layout_passes=False)


@functools.partial(
    pl.kernel,
    mesh=_mesh,
    out_type=jax.ShapeDtypeStruct((NC, NUM_LEVELS), jnp.float32),
    scratch_types=[
        pltpu.VMEM((BUF,), jnp.float32),
        pltpu.VMEM((BUF,), jnp.float32),
        pltpu.VMEM((NUM_LEVELS,), jnp.float32),
        pltpu.VMEM((HBINS,), jnp.float32),
        pltpu.VMEM((HBINS,), jnp.float32),
        pltpu.VMEM_SHARED((NS, NUM_LEVELS), jnp.float32),
        pltpu.SemaphoreType.DMA,
        pltpu.SemaphoreType.DMA,
    ],
    compiler_params=_params,
)
def _hist_kernel(vals_hbm, part_hbm, buf0, buf1, hist, tmp, acc, shared,
                 sem0, sem1):
    c = lax.axis_index("c")
    s = lax.axis_index("s")
    wid = s * NC + c
    base = wid * E

    bufs = (buf0, buf1)
    sems = (sem0, sem1)

    zeros16 = jnp.zeros((L,), jnp.float32)
    ones16 = jnp.ones((L,), jnp.float32)
    tailmask = lax.iota(jnp.int32, L) < TAIL

    buf0[pl.ds(BUF - L, L)] = zeros16
    buf1[pl.ds(BUF - L, L)] = zeros16

    copies = [None] * NCHUNK
    copies[0] = pltpu.async_copy(vals_hbm.at[pl.ds(base, CH)],
                                 buf0.at[pl.ds(0, CH)], sem0)

    @plsc.parallel_loop(0, HVECS, unroll=UNROLL)
    def _(i):
        hist[pl.ds(i * L, L)] = zeros16

    for ch in range(NCHUNK):
        if ch + 1 < NCHUNK:
            nxt = (ch + 1) % 2
            copies[ch + 1] = pltpu.async_copy(
                vals_hbm.at[pl.ds(base + (ch + 1) * CH, CH)],
                bufs[nxt].at[pl.ds(0, CH)], sems[nxt])
        copies[ch].wait()
        buf = bufs[ch % 2]

        @plsc.parallel_loop(0, FULL_VECS, unroll=UNROLL)
        def _(i):
            idx = buf[pl.ds(i * L, L)].astype(jnp.int32)
            plsc.addupdate_scatter(hist, [idx], ones16)

        idx = buf[pl.ds(FULL_VECS * L, L)].astype(jnp.int32)
        plsc.addupdate_scatter(hist, [idx], ones16, mask=tailmask)

    pltpu.sync_copy(hist, shared.at[s])
    plsc.subcore_barrier()

    @plsc.parallel_loop(0, HBINS // L, unroll=4)
    def _(i):
        acc[pl.ds(i * L, L)] = zeros16

    def row_body(j, carry):
        pltpu.sync_copy(shared.at[j, pl.ds(s * HBINS, HBINS)], tmp)

        @plsc.parallel_loop(0, HBINS // L, unroll=4)
        def _(k):
            sl = pl.ds(k * L, L)
            acc[sl] = acc[sl] + tmp[sl]

        return carry

    lax.fori_loop(0, NS, row_body, 0)
    pltpu.sync_copy(acc, part_hbm.at[c, pl.ds(s * HBINS, HBINS)])


@functools.partial(
    pl.kernel,
    mesh=_mesh,
    out_type=jax.ShapeDtypeStruct((N,), jnp.float32),
    scratch_types=[
        pltpu.VMEM((BUF,), jnp.float32),
        pltpu.VMEM((BUF,), jnp.float32),
        pltpu.VMEM((BUF,), jnp.float32),
        pltpu.VMEM((BUF,), jnp.float32),
        pltpu.VMEM((NUM_LEVELS,), jnp.float32),
        pltpu.VMEM((NUM_LEVELS,), jnp.float32),
        pltpu.SemaphoreType.DMA,
        pltpu.SemaphoreType.DMA,
        pltpu.SemaphoreType.DMA,
        pltpu.SemaphoreType.DMA,
    ],
    compiler_params=_params,
)
def _lookup_kernel(part_hbm, vals_hbm, out_hbm, buf0, buf1, obuf0, obuf1,
                   hist, h2, isem0, isem1, osem0, osem1):
    c = lax.axis_index("c")
    s = lax.axis_index("s")
    wid = s * NC + c
    base = wid * E

    bufs = (buf0, buf1)
    obufs = (obuf0, obuf1)
    isems = (isem0, isem1)
    osems = (osem0, osem1)

    zeros16 = jnp.zeros((L,), jnp.float32)
    buf0[pl.ds(BUF - L, L)] = zeros16
    buf1[pl.ds(BUF - L, L)] = zeros16

    in_copies = [None] * NCHUNK
    in_copies[0] = pltpu.async_copy(vals_hbm.at[pl.ds(base, CH)],
                                    buf0.at[pl.ds(0, CH)], isem0)
    if NCHUNK > 1:
        in_copies[1] = pltpu.async_copy(vals_hbm.at[pl.ds(base + CH, CH)],
                                        buf1.at[pl.ds(0, CH)], isem1)

    pltpu.sync_copy(part_hbm.at[0], hist)
    pltpu.sync_copy(part_hbm.at[1], h2)

    @plsc.parallel_loop(0, HVECS, unroll=4, carry=zeros16)
    def addmax_body(i, m):
        sl = pl.ds(i * L, L)
        hv = hist[sl] + h2[sl]
        hist[sl] = hv
        return jnp.maximum(m, hv)

    mx = lax.broadcast(jnp.max(addmax_body), (L,))
    recip = jnp.ones((L,), jnp.float32) / mx

    out_copies = [None] * NCHUNK
    for ch in range(NCHUNK):
        in_copies[ch].wait()
        buf = bufs[ch % 2]
        obuf = obufs[ch % 2]
        if ch >= 2:
            out_copies[ch - 2].wait()

        @plsc.parallel_loop(0, FULL_VECS + 1, unroll=UNROLL)
        def _(i):
            sl = pl.ds(i * L, L)
            idx = buf[sl].astype(jnp.int32)
            obuf[sl] = plsc.load_gather(hist, [idx]) * recip

        if ch + 2 < NCHUNK:
            in_copies[ch + 2] = pltpu.async_copy(
                vals_hbm.at[pl.ds(base + (ch + 2) * CH, CH)],
                bufs[ch % 2].at[pl.ds(0, CH)], isems[ch % 2])

        out_copies[ch] = pltpu.async_copy(
            obuf.at[pl.ds(0, CH)],
            out_hbm.at[pl.ds(base + ch * CH, CH)], osems[ch % 2])

    for ch in range(max(0, NCHUNK - 2), NCHUNK):
        out_copies[ch].wait()


def kernel(node_values):
    part = _hist_kernel(node_values)
    return _lookup_kernel(part, node_values)


# speedup vs baseline: 347.0282x; 1.0069x over previous
import functools

import jax
import jax.numpy as jnp
from jax import lax
from jax.experimental import pallas as pl
from jax.experimental.pallas import tpu as pltpu
from jax.experimental.pallas import tpu_sc as plsc

N = 4_000_000
NUM_LEVELS = 4096
L = 16
NC = 2
NS = 16
NW = NC * NS
E = N // NW
CH = 25_000
NCHUNK = E // CH
FULL_VECS = CH // L
TAIL = CH - FULL_VECS * L
BUF = (CH + L - 1) // L * L
HBINS = NUM_LEVELS // NS
HVECS = NUM_LEVELS // L
UNROLL = 8

_mesh = plsc.VectorSubcoreMesh(core_axis_name="c", subcore_axis_name="s")
_params = pltpu.CompilerParams(needs_layout_passes=False)


@functools.partial(
    pl.kernel,
    mesh=_mesh,
    out_type=jax.ShapeDtypeStruct((NC, NUM_LEVELS), jnp.float32),
    scratch_types=[
        pltpu.VMEM((BUF,), jnp.float32),
        pltpu.VMEM((BUF,), jnp.float32),
        pltpu.VMEM((BUF,), jnp.float32),
        pltpu.VMEM((NUM_LEVELS,), jnp.float32),
        pltpu.VMEM((NS, HBINS), jnp.float32),
        pltpu.VMEM((HBINS,), jnp.float32),
        pltpu.VMEM_SHARED((NS, NUM_LEVELS), jnp.float32),
        pltpu.SemaphoreType.DMA,
        pltpu.SemaphoreType.DMA,
        pltpu.SemaphoreType.DMA,
    ],
    compiler_params=_params,
)
def _hist_kernel(vals_hbm, part_hbm, buf0, buf1, buf2, hist, tmp2d, acc,
                 shared, sem0, sem1, sem2):
    c = lax.axis_index("c")
    s = lax.axis_index("s")
    wid = s * NC + c
    base = wid * E

    bufs = (buf0, buf1, buf2)
    sems = (sem0, sem1, sem2)

    zeros16 = jnp.zeros((L,), jnp.float32)
    ones16 = jnp.ones((L,), jnp.float32)
    tailmask = lax.iota(jnp.int32, L) < TAIL

    for b in bufs:
        b[pl.ds(BUF - L, L)] = zeros16

    copies = [None] * NCHUNK
    for ch in range(min(3, NCHUNK)):
        copies[ch] = pltpu.async_copy(
            vals_hbm.at[pl.ds(base + ch * CH, CH)],
            bufs[ch].at[pl.ds(0, CH)], sems[ch])

    @plsc.parallel_loop(0, HVECS, unroll=UNROLL)
    def _(i):
        hist[pl.ds(i * L, L)] = zeros16

    for ch in range(NCHUNK):
        copies[ch].wait()
        buf = bufs[ch % 3]

        @plsc.parallel_loop(0, FULL_VECS, unroll=UNROLL)
        def _(i):
            idx = buf[pl.ds(i * L, L)].astype(jnp.int32)
            plsc.addupdate_scatter(hist, [idx], ones16)

        idx = buf[pl.ds(FULL_VECS * L, L)].astype(jnp.int32)
        plsc.addupdate_scatter(hist, [idx], ones16, mask=tailmask)

        if ch + 3 < NCHUNK:
            copies[ch + 3] = pltpu.async_copy(
                vals_hbm.at[pl.ds(base + (ch + 3) * CH, CH)],
                bufs[ch % 3].at[pl.ds(0, CH)], sems[ch % 3])

    pltpu.sync_copy(hist, shared.at[s])
    plsc.subcore_barrier()
    red_copies = [
        pltpu.async_copy(shared.at[j, pl.ds(s * HBINS, HBINS)],
                         tmp2d.at[j], sem0)
        for j in range(NS)
    ]
    for cp in red_copies:
        cp.wait()

    @plsc.parallel_loop(0, HBINS // L, unroll=4)
    def _(k):
        sl = pl.ds(k * L, L)
        v = tmp2d[0, sl]
        for j in range(1, NS):
            v = v + tmp2d[j, sl]
        acc[sl] = v

    pltpu.sync_copy(acc, part_hbm.at[c, pl.ds(s * HBINS, HBINS)])


@functools.partial(
    pl.kernel,
    mesh=_mesh,
    out_type=jax.ShapeDtypeStruct((N,), jnp.float32),
    scratch_types=[
        pltpu.VMEM((BUF,), jnp.float32),
        pltpu.VMEM((BUF,), jnp.float32),
        pltpu.VMEM((BUF,), jnp.float32),
        pltpu.VMEM((BUF,), jnp.float32),
        pltpu.VMEM((NUM_LEVELS,), jnp.float32),
        pltpu.VMEM((NC, NUM_LEVELS), jnp.float32),
        pltpu.SemaphoreType.DMA,
        pltpu.SemaphoreType.DMA,
        pltpu.SemaphoreType.DMA,
        pltpu.SemaphoreType.DMA,
    ],
    compiler_params=_params,
)
def _lookup_kernel(part_hbm, vals_hbm, out_hbm, buf0, buf1, obuf0, obuf1,
                   hist, h2d, isem0, isem1, osem0, osem1):
    c = lax.axis_index("c")
    s = lax.axis_index("s")
    wid = s * NC + c
    base = wid * E

    bufs = (buf0, buf1)
    obufs = (obuf0, obuf1)
    isems = (isem0, isem1)
    osems = (osem0, osem1)

    zeros16 = jnp.zeros((L,), jnp.float32)
    buf0[pl.ds(BUF - L, L)] = zeros16
    buf1[pl.ds(BUF - L, L)] = zeros16

    in_copies = [None] * NCHUNK
    in_copies[0] = pltpu.async_copy(vals_hbm.at[pl.ds(base, CH)],
                                    buf0.at[pl.ds(0, CH)], isem0)
    if NCHUNK > 1:
        in_copies[1] = pltpu.async_copy(vals_hbm.at[pl.ds(base + CH, CH)],
                                        buf1.at[pl.ds(0, CH)], isem1)

    pltpu.sync_copy(part_hbm, h2d)

    @plsc.parallel_loop(0, HVECS, unroll=4, carry=zeros16)
    def addmax_body(i, m):
        sl = pl.ds(i * L, L)
        hv = h2d[0, sl] + h2d[1, sl]
        hist[sl] = hv
        return jnp.maximum(m, hv)

    mx = lax.broadcast(jnp.max(addmax_body), (L,))
    recip = jnp.ones((L,), jnp.float32) / mx

    out_copies = [None] * NCHUNK
    for ch in range(NCHUNK):
        in_copies[ch].wait()
        buf = bufs[ch % 2]
        obuf = obufs[ch % 2]
        if ch >= 2:
            out_copies[ch - 2].wait()

        @plsc.parallel_loop(0, FULL_VECS + 1, unroll=UNROLL)
        def _(i):
            sl = pl.ds(i * L, L)
            idx = buf[sl].astype(jnp.int32)
            obuf[sl] = plsc.load_gather(hist, [idx]) * recip

        if ch + 2 < NCHUNK:
            in_copies[ch + 2] = pltpu.async_copy(
                vals_hbm.at[pl.ds(base + (ch + 2) * CH, CH)],
                bufs[ch % 2].at[pl.ds(0, CH)], isems[ch % 2])

        out_copies[ch] = pltpu.async_copy(
            obuf.at[pl.ds(0, CH)],
            out_hbm.at[pl.ds(base + ch * CH, CH)], osems[ch % 2])

    for ch in range(max(0, NCHUNK - 2), NCHUNK):
        out_copies[ch].wait()


def kernel(node_values):
    part = _hist_kernel(node_values)
    return _lookup_kernel(part, node_values)
